# Initial kernel scaffold; baseline (speedup 1.0000x reference)
import jax
import jax.numpy as jnp
from jax.experimental import pallas as pl


def kernel(x, tables):
    raise NotImplementedError("write your pallas kernel here")


# speedup vs baseline: 58.0600x; 58.0600x over previous
import functools

import numpy as np
import jax
import jax.numpy as jnp
from jax import lax
from jax.experimental import pallas as pl
from jax.experimental.pallas import tpu as pltpu
from jax.experimental.pallas import tpu_sc as plsc

N_POINTS = 1048576
N_LEVELS = 16
N_FEATS = 2
LOG2_HASHMAP_SIZE = 19
TABLE_SIZE = 1 << LOG2_HASHMAP_SIZE
BASE_RES = 16.0
FINEST_RES = 512.0

NUM_CORES = 2
NUM_SUBCORES = 16
NUM_WORKERS = NUM_CORES * NUM_SUBCORES
P_PER_W = N_POINTS // NUM_WORKERS
CHUNK = 512
N_CHUNKS = P_PER_W // CHUNK
GROUPS = CHUNK // 128
G_PER_LV = 8 * GROUPS

_B_GROWTH = np.exp((np.log(FINEST_RES) - np.log(BASE_RES)) / (N_LEVELS - 1))
_GS = [np.float32(1.0) / np.float32(float(np.floor(BASE_RES * _B_GROWTH ** i)))
       for i in range(N_LEVELS)]

_P1 = np.uint32(2654435761)
_P2 = np.uint32(805459861)
_HMASK = np.uint32(TABLE_SIZE - 1)


def _make_kernel():
    mesh = plsc.VectorSubcoreMesh(core_axis_name="c", subcore_axis_name="s")

    scratch = [
        pltpu.VMEM((CHUNK,), jnp.float32),
        pltpu.VMEM((CHUNK,), jnp.float32),
        pltpu.VMEM((CHUNK,), jnp.float32),
        [[pltpu.VMEM((CHUNK,), jnp.float32) for _ in range(3)]
         for _ in range(2)],
        [pltpu.VMEM((8 * CHUNK,), jnp.int32) for _ in range(2)],
        [pltpu.VMEM((8 * CHUNK, N_FEATS), jnp.float32)
         for _ in range(2)],
        pltpu.VMEM((CHUNK, 2 * N_LEVELS), jnp.float32),
        [pltpu.SemaphoreType.DMA for _ in range(2)],
    ]

    @functools.partial(
        pl.kernel,
        out_type=jax.ShapeDtypeStruct((N_POINTS, 2 * N_LEVELS), jnp.float32),
        mesh=mesh,
        scratch_types=scratch,
        compiler_params=pltpu.CompilerParams(
            needs_layout_passes=False, use_tc_tiling_on_sc=False),
    )
    def hash_embed(x0h, x1h, x2h, tabh, outh,
                   x0v, x1v, x2v, wv, idxv, rowsv, outv, semg):
        cid = lax.axis_index("c")
        sid = lax.axis_index("s")
        wid = sid * NUM_CORES + cid

        iota = lax.iota(jnp.int32, 16)
        half = iota >> 1
        parity = iota & 1

        def hash_pass(lv):
            buf = lv & 1
            gs = _GS[lv]
            lvoff = np.uint32(lv << LOG2_HASHMAP_SIZE)
            w0v, w1v, w2v = wv[buf]
            idx = idxv[buf]

            def body(i, carry):
                s = i * 16
                q0 = x0v[pl.ds(s, 16)] / gs
                q1 = x1v[pl.ds(s, 16)] / gs
                q2 = x2v[pl.ds(s, 16)] / gs
                b0 = q0.astype(jnp.int32)
                b1 = q1.astype(jnp.int32)
                b2 = q2.astype(jnp.int32)
                w0v[pl.ds(s, 16)] = q0 - b0.astype(jnp.float32)
                w1v[pl.ds(s, 16)] = q1 - b1.astype(jnp.float32)
                w2v[pl.ds(s, 16)] = q2 - b2.astype(jnp.float32)
                a0 = b0.astype(jnp.uint32)
                m1 = b1.astype(jnp.uint32) * _P1
                m2 = b2.astype(jnp.uint32) * _P2
                a0p = a0 + np.uint32(1)
                m1p = m1 + _P1
                m2p = m2 + _P2
                xy00 = a0 ^ m1
                xy01 = a0 ^ m1p
                xy10 = a0p ^ m1
                xy11 = a0p ^ m1p
                for c, (xy, mz) in enumerate((
                        (xy00, m2), (xy00, m2p), (xy01, m2), (xy01, m2p),
                        (xy10, m2), (xy10, m2p), (xy11, m2), (xy11, m2p))):
                    h = ((xy ^ mz) & _HMASK) | lvoff
                    idx[pl.ds(c * CHUNK + s, 16)] = plsc.bitcast(
                        h, jnp.int32)
                return carry

            lax.fori_loop(0, CHUNK // 16, body, 0, unroll=2)

        def fire(lv):
            buf = lv & 1
            return pltpu.async_copy(
                tabh.at[idxv[buf]], rowsv[buf], semg[buf])

        def interp(lv):
            buf = lv & 1
            w0v, w1v, w2v = wv[buf]
            rows = rowsv[buf]
            col = 2 * lv + parity

            def body(j, carry):
                dup = j * 8 + half
                wd0 = plsc.load_gather(w0v, [dup])
                wd1 = plsc.load_gather(w1v, [dup])
                wd2 = plsc.load_gather(w2v, [dup])
                u0 = 1.0 - wd0
                u1 = 1.0 - wd1
                u2 = 1.0 - wd2
                e = []
                for c in range(8):
                    e.append(plsc.load_gather(
                        rows, [c * CHUNK + dup, parity]))
                c00 = e[0] * u0 + e[4] * wd0
                c01 = e[1] * u0 + e[5] * wd0
                c10 = e[2] * u0 + e[6] * wd0
                c11 = e[3] * u0 + e[7] * wd0
                c0 = c00 * u1 + c10 * wd1
                c1 = c01 * u1 + c11 * wd1
                cc = c0 * u2 + c1 * wd2
                row = j * 8 + half
                plsc.store_scatter(outv, [row, col], cc)
                return carry

            lax.fori_loop(0, CHUNK // 8, body, 0, unroll=2)

        def run_chunk(ch, carry):
            base = wid * P_PER_W + ch * CHUNK
            pltpu.sync_copy(x0h.at[pl.ds(base, CHUNK)], x0v)
            pltpu.sync_copy(x1h.at[pl.ds(base, CHUNK)], x1v)
            pltpu.sync_copy(x2h.at[pl.ds(base, CHUNK)], x2v)

            hash_pass(0)
            handle = fire(0)
            for lv in range(N_LEVELS):
                nxt = None
                if lv + 1 < N_LEVELS:
                    hash_pass(lv + 1)
                    nxt = fire(lv + 1)
                handle.wait()
                interp(lv)
                handle = nxt
            pltpu.sync_copy(outv, outh.at[pl.ds(base, CHUNK)])
            return carry

        lax.fori_loop(0, N_CHUNKS, run_chunk, 0)

    return hash_embed


_hash_embed = _make_kernel()


def kernel(x, tables):
    xt = x.T
    out = _hash_embed(xt[0], xt[1], xt[2],
                      tables.reshape(N_LEVELS * TABLE_SIZE, N_FEATS))
    keep_mask = jnp.ones((N_POINTS,), dtype=jnp.bool_)
    return out, keep_mask


# speedup vs baseline: 58.1385x; 1.0014x over previous
import functools

import numpy as np
import jax
import jax.numpy as jnp
from jax import lax
from jax.experimental import pallas as pl
from jax.experimental.pallas import tpu as pltpu
from jax.experimental.pallas import tpu_sc as plsc

N_POINTS = 1048576
N_LEVELS = 16
N_FEATS = 2
LOG2_HASHMAP_SIZE = 19
TABLE_SIZE = 1 << LOG2_HASHMAP_SIZE
BASE_RES = 16.0
FINEST_RES = 512.0

NUM_CORES = 2
NUM_SUBCORES = 16
NUM_WORKERS = NUM_CORES * NUM_SUBCORES
P_PER_W = N_POINTS // NUM_WORKERS
CHUNK = 512
N_CHUNKS = P_PER_W // CHUNK
GROUPS = CHUNK // 128
G_PER_LV = 8 * GROUPS

_B_GROWTH = np.exp((np.log(FINEST_RES) - np.log(BASE_RES)) / (N_LEVELS - 1))
_GS = [np.float32(1.0) / np.float32(float(np.floor(BASE_RES * _B_GROWTH ** i)))
       for i in range(N_LEVELS)]

_P1 = np.uint32(2654435761)
_P2 = np.uint32(805459861)
_HMASK = np.uint32(TABLE_SIZE - 1)


def _make_kernel():
    mesh = plsc.VectorSubcoreMesh(core_axis_name="c", subcore_axis_name="s")

    scratch = [
        [pltpu.VMEM((CHUNK,), jnp.float32) for _ in range(3)],
        [[pltpu.VMEM((CHUNK,), jnp.float32) for _ in range(3)]
         for _ in range(2)],
        [pltpu.VMEM((8 * CHUNK,), jnp.int32) for _ in range(2)],
        [pltpu.VMEM((8 * CHUNK, N_FEATS), jnp.float32)
         for _ in range(2)],
        pltpu.VMEM((CHUNK, 2 * N_LEVELS), jnp.float32),
        [pltpu.SemaphoreType.DMA for _ in range(2)],
    ]

    @functools.partial(
        pl.kernel,
        out_type=jax.ShapeDtypeStruct((N_POINTS, 2 * N_LEVELS), jnp.float32),
        mesh=mesh,
        scratch_types=scratch,
        compiler_params=pltpu.CompilerParams(
            needs_layout_passes=False, use_tc_tiling_on_sc=False),
    )
    def hash_embed(x0h, x1h, x2h, tabh, outh,
                   xv, wv, idxv, rowsv, outv, semg):
        cid = lax.axis_index("c")
        sid = lax.axis_index("s")
        wid = sid * NUM_CORES + cid

        iota = lax.iota(jnp.int32, 16)
        half = iota >> 1
        parity = iota & 1
        x0v, x1v, x2v = xv

        def hash_pass(lv):
            buf = lv & 1
            gs = _GS[lv]
            lvoff = np.uint32(lv << LOG2_HASHMAP_SIZE)
            w0v, w1v, w2v = wv[buf]
            idx = idxv[buf]

            def body(i, carry):
                s = i * 16
                q0 = x0v[pl.ds(s, 16)] / gs
                q1 = x1v[pl.ds(s, 16)] / gs
                q2 = x2v[pl.ds(s, 16)] / gs
                b0 = q0.astype(jnp.int32)
                b1 = q1.astype(jnp.int32)
                b2 = q2.astype(jnp.int32)
                w0v[pl.ds(s, 16)] = q0 - b0.astype(jnp.float32)
                w1v[pl.ds(s, 16)] = q1 - b1.astype(jnp.float32)
                w2v[pl.ds(s, 16)] = q2 - b2.astype(jnp.float32)
                a0 = b0.astype(jnp.uint32)
                m1 = b1.astype(jnp.uint32) * _P1
                m2 = b2.astype(jnp.uint32) * _P2
                a0p = a0 + np.uint32(1)
                m1p = m1 + _P1
                m2p = m2 + _P2
                xy00 = a0 ^ m1
                xy01 = a0 ^ m1p
                xy10 = a0p ^ m1
                xy11 = a0p ^ m1p
                for c, (xy, mz) in enumerate((
                        (xy00, m2), (xy00, m2p), (xy01, m2), (xy01, m2p),
                        (xy10, m2), (xy10, m2p), (xy11, m2), (xy11, m2p))):
                    h = ((xy ^ mz) & _HMASK) | lvoff
                    idx[pl.ds(c * CHUNK + s, 16)] = plsc.bitcast(
                        h, jnp.int32)
                return carry

            lax.fori_loop(0, CHUNK // 16, body, 0, unroll=2)

        def fire(lv):
            buf = lv & 1
            return pltpu.async_copy(
                tabh.at[idxv[buf]], rowsv[buf], semg[buf])

        def interp(lv):
            buf = lv & 1
            w0v, w1v, w2v = wv[buf]
            rows = rowsv[buf]
            col = 2 * lv + parity

            def body(j, carry):
                dup = j * 8 + half
                wd0 = plsc.load_gather(w0v, [dup])
                wd1 = plsc.load_gather(w1v, [dup])
                wd2 = plsc.load_gather(w2v, [dup])
                u0 = 1.0 - wd0
                u1 = 1.0 - wd1
                u2 = 1.0 - wd2
                e = []
                for c in range(8):
                    e.append(plsc.load_gather(
                        rows, [c * CHUNK + dup, parity]))
                c00 = e[0] * u0 + e[4] * wd0
                c01 = e[1] * u0 + e[5] * wd0
                c10 = e[2] * u0 + e[6] * wd0
                c11 = e[3] * u0 + e[7] * wd0
                c0 = c00 * u1 + c10 * wd1
                c1 = c01 * u1 + c11 * wd1
                cc = c0 * u2 + c1 * wd2
                row = j * 8 + half
                plsc.store_scatter(outv, [row, col], cc)
                return carry

            lax.fori_loop(0, CHUNK // 8, body, 0, unroll=2)

        def run_chunk(ch, carry):
            base = wid * P_PER_W + ch * CHUNK
            pltpu.sync_copy(x0h.at[pl.ds(base, CHUNK)], x0v)
            pltpu.sync_copy(x1h.at[pl.ds(base, CHUNK)], x1v)
            pltpu.sync_copy(x2h.at[pl.ds(base, CHUNK)], x2v)

            hash_pass(0)
            handle = fire(0)
            for lv in range(N_LEVELS):
                nxt = None
                if lv + 1 < N_LEVELS:
                    hash_pass(lv + 1)
                    nxt = fire(lv + 1)
                handle.wait()
                interp(lv)
                handle = nxt
            pltpu.sync_copy(outv, outh.at[pl.ds(base, CHUNK)])
            return carry

        lax.fori_loop(0, N_CHUNKS, run_chunk, 0)

    return hash_embed


_hash_embed = _make_kernel()


def kernel(x, tables):
    xt = x.T
    out = _hash_embed(xt[0], xt[1], xt[2],
                      tables.reshape(N_LEVELS * TABLE_SIZE, N_FEATS))
    keep_mask = jnp.ones((N_POINTS,), dtype=jnp.bool_)
    return out, keep_mask


# speedup vs baseline: 58.3845x; 1.0042x over previous
import functools

import numpy as np
import jax
import jax.numpy as jnp
from jax import lax
from jax.experimental import pallas as pl
from jax.experimental.pallas import tpu as pltpu
from jax.experimental.pallas import tpu_sc as plsc

N_POINTS = 1048576
N_LEVELS = 16
N_FEATS = 2
LOG2_HASHMAP_SIZE = 19
TABLE_SIZE = 1 << LOG2_HASHMAP_SIZE
BASE_RES = 16.0
FINEST_RES = 512.0

NUM_CORES = 2
NUM_SUBCORES = 16
NUM_WORKERS = NUM_CORES * NUM_SUBCORES
P_PER_W = N_POINTS // NUM_WORKERS
CHUNK = 512
N_CHUNKS = P_PER_W // CHUNK
GROUPS = CHUNK // 128
G_PER_LV = 8 * GROUPS

_B_GROWTH = np.exp((np.log(FINEST_RES) - np.log(BASE_RES)) / (N_LEVELS - 1))
_GS = [np.float32(1.0) / np.float32(float(np.floor(BASE_RES * _B_GROWTH ** i)))
       for i in range(N_LEVELS)]

_P1 = np.uint32(2654435761)
_P2 = np.uint32(805459861)
_HMASK = np.uint32(TABLE_SIZE - 1)


def _make_kernel():
    mesh = plsc.VectorSubcoreMesh(core_axis_name="c", subcore_axis_name="s")

    scratch = [
        [pltpu.VMEM((CHUNK,), jnp.float32) for _ in range(3)],
        [[pltpu.VMEM((CHUNK,), jnp.float32) for _ in range(3)]
         for _ in range(2)],
        [pltpu.VMEM((8 * CHUNK,), jnp.int32) for _ in range(2)],
        [pltpu.VMEM((8 * CHUNK, 8), jnp.float32)
         for _ in range(2)],
        pltpu.VMEM((CHUNK, 2 * N_LEVELS), jnp.float32),
        [pltpu.SemaphoreType.DMA for _ in range(2)],
    ]

    @functools.partial(
        pl.kernel,
        out_type=jax.ShapeDtypeStruct((N_POINTS, 2 * N_LEVELS), jnp.float32),
        mesh=mesh,
        scratch_types=scratch,
        compiler_params=pltpu.CompilerParams(
            needs_layout_passes=False, use_tc_tiling_on_sc=False),
    )
    def hash_embed(x0h, x1h, x2h, tabh, outh,
                   xv, wv, idxv, rowsv, outv, semg):
        cid = lax.axis_index("c")
        sid = lax.axis_index("s")
        wid = sid * NUM_CORES + cid

        iota = lax.iota(jnp.int32, 16)
        half = iota >> 1
        parity = iota & 1
        x0v, x1v, x2v = xv

        def hash_pass(lv):
            buf = lv & 1
            gs = _GS[lv]
            lvoff = np.uint32(lv << LOG2_HASHMAP_SIZE)
            w0v, w1v, w2v = wv[buf]
            idx = idxv[buf]

            def body(i, carry):
                s = i * 16
                q0 = x0v[pl.ds(s, 16)] / gs
                q1 = x1v[pl.ds(s, 16)] / gs
                q2 = x2v[pl.ds(s, 16)] / gs
                b0 = q0.astype(jnp.int32)
                b1 = q1.astype(jnp.int32)
                b2 = q2.astype(jnp.int32)
                w0v[pl.ds(s, 16)] = q0 - b0.astype(jnp.float32)
                w1v[pl.ds(s, 16)] = q1 - b1.astype(jnp.float32)
                w2v[pl.ds(s, 16)] = q2 - b2.astype(jnp.float32)
                a0 = b0.astype(jnp.uint32)
                m1 = b1.astype(jnp.uint32) * _P1
                m2 = b2.astype(jnp.uint32) * _P2
                a0p = a0 + np.uint32(1)
                m1p = m1 + _P1
                m2p = m2 + _P2
                xy00 = a0 ^ m1
                xy01 = a0 ^ m1p
                xy10 = a0p ^ m1
                xy11 = a0p ^ m1p
                for c, (xy, mz) in enumerate((
                        (xy00, m2), (xy00, m2p), (xy01, m2), (xy01, m2p),
                        (xy10, m2), (xy10, m2p), (xy11, m2), (xy11, m2p))):
                    h = ((xy ^ mz) & _HMASK) | lvoff
                    idx[pl.ds(c * CHUNK + s, 16)] = plsc.bitcast(
                        h, jnp.int32)
                return carry

            lax.fori_loop(0, CHUNK // 16, body, 0, unroll=2)

        def fire(lv):
            buf = lv & 1
            return pltpu.async_copy(
                tabh.at[idxv[buf]], rowsv[buf], semg[buf])

        def interp(lv):
            buf = lv & 1
            w0v, w1v, w2v = wv[buf]
            rows = rowsv[buf]
            col = 2 * lv + parity

            def body(j, carry):
                dup = j * 8 + half
                wd0 = plsc.load_gather(w0v, [dup])
                wd1 = plsc.load_gather(w1v, [dup])
                wd2 = plsc.load_gather(w2v, [dup])
                u0 = 1.0 - wd0
                u1 = 1.0 - wd1
                u2 = 1.0 - wd2
                e = []
                for c in range(8):
                    e.append(plsc.load_gather(
                        rows, [c * CHUNK + dup, parity]))
                c00 = e[0] * u0 + e[4] * wd0
                c01 = e[1] * u0 + e[5] * wd0
                c10 = e[2] * u0 + e[6] * wd0
                c11 = e[3] * u0 + e[7] * wd0
                c0 = c00 * u1 + c10 * wd1
                c1 = c01 * u1 + c11 * wd1
                cc = c0 * u2 + c1 * wd2
                row = j * 8 + half
                plsc.store_scatter(outv, [row, col], cc)
                return carry

            lax.fori_loop(0, CHUNK // 8, body, 0, unroll=2)

        def run_chunk(ch, carry):
            base = wid * P_PER_W + ch * CHUNK
            pltpu.sync_copy(x0h.at[pl.ds(base, CHUNK)], x0v)
            pltpu.sync_copy(x1h.at[pl.ds(base, CHUNK)], x1v)
            pltpu.sync_copy(x2h.at[pl.ds(base, CHUNK)], x2v)

            hash_pass(0)
            handle = fire(0)
            for lv in range(N_LEVELS):
                nxt = None
                if lv + 1 < N_LEVELS:
                    hash_pass(lv + 1)
                    nxt = fire(lv + 1)
                handle.wait()
                interp(lv)
                handle = nxt
            pltpu.sync_copy(outv, outh.at[pl.ds(base, CHUNK)])
            return carry

        lax.fori_loop(0, N_CHUNKS, run_chunk, 0)

    return hash_embed


_hash_embed = _make_kernel()


def kernel(x, tables):
    xt = x.T
    tab8 = jnp.pad(tables.reshape(N_LEVELS * TABLE_SIZE, N_FEATS),
                   ((0, 0), (0, 8 - N_FEATS)))
    out = _hash_embed(xt[0], xt[1], xt[2], tab8)
    keep_mask = jnp.ones((N_POINTS,), dtype=jnp.bool_)
    return out, keep_mask
